# Initial kernel scaffold; baseline (speedup 1.0000x reference)
import jax
import jax.numpy as jnp
from jax.experimental import pallas as pl


def kernel(x, gate_w, w1, w2, w3, sw1, sw2, sw3):
    raise NotImplementedError("write your pallas kernel here")


# speedup vs baseline: 2.1951x; 2.1951x over previous
import functools

import jax
import jax.numpy as jnp
from jax import lax
from jax.experimental import pallas as pl
from jax.experimental.pallas import tpu as pltpu

DIM = 1024
INTER = 512
E = 16
T = 2048
BM = 256
MT = T // BM


def _silu(v):
    return v * jax.nn.sigmoid(v)


def _bdot(a, b):
    return lax.dot_general(a, b, (((1,), (1,)), ((), ())),
                           preferred_element_type=jnp.float32)


def _moe_body(xr, gwr, w1r, w2r, w3r, sw1r, sw2r, sw3r, outr, cw_s, xb_s):
    e = pl.program_id(0)
    m = pl.program_id(1)
    rows = pl.ds(m * BM, BM)

    @pl.when(e == 0)
    def _init():
        xf = xr[0, rows, :]
        xb_s[rows, :] = xf.astype(jnp.bfloat16)
        logits = _bdot(xf, gwr[...])
        mx = jnp.max(logits, axis=1, keepdims=True)
        p = jnp.exp(logits - mx)
        p = p / jnp.sum(p, axis=1, keepdims=True)
        iota = lax.broadcasted_iota(jnp.int32, (BM, E), 1)
        m1 = jnp.max(p, axis=1, keepdims=True)
        i1 = jnp.min(jnp.where(p == m1, iota, E), axis=1, keepdims=True)
        p2 = jnp.where(iota == i1, -1.0, p)
        m2 = jnp.max(p2, axis=1, keepdims=True)
        i2 = jnp.min(jnp.where(p2 == m2, iota, E), axis=1, keepdims=True)
        cw_s[rows, :] = (jnp.where(iota == i1, m1, 0.0)
                         + jnp.where(iota == i2, m2, 0.0))
        xb = xb_s[rows, :]
        h1 = _bdot(xb, sw1r[...].astype(jnp.bfloat16))
        h3 = _bdot(xb, sw3r[...].astype(jnp.bfloat16))
        hh = (_silu(h1) * h3).astype(jnp.bfloat16)
        outr[0, rows, :] = _bdot(hh, sw2r[...].astype(jnp.bfloat16))

    xb = xb_s[rows, :]
    h1 = _bdot(xb, w1r[0].astype(jnp.bfloat16))
    h3 = _bdot(xb, w3r[0].astype(jnp.bfloat16))
    iota = lax.broadcasted_iota(jnp.int32, (BM, E), 1)
    cwcol = jnp.sum(jnp.where(iota == e, cw_s[rows, :], 0.0), axis=1,
                    keepdims=True)
    hh = (_silu(h1) * h3 * cwcol).astype(jnp.bfloat16)
    outr[0, rows, :] += _bdot(hh, w2r[0].astype(jnp.bfloat16))


@functools.partial(jax.jit, static_argnames=("interpret",))
def _moe(x, gate_w, w1, w2, w3, sw1, sw2, sw3, interpret=False):
    out = pl.pallas_call(
        _moe_body,
        grid=(E, MT),
        in_specs=[
            pl.BlockSpec((1, T, DIM), lambda e, m: (0, 0, 0)),
            pl.BlockSpec((E, DIM), lambda e, m: (0, 0)),
            pl.BlockSpec((1, INTER, DIM), lambda e, m: (e, 0, 0)),
            pl.BlockSpec((1, DIM, INTER), lambda e, m: (e, 0, 0)),
            pl.BlockSpec((1, INTER, DIM), lambda e, m: (e, 0, 0)),
            pl.BlockSpec((2 * INTER, DIM), lambda e, m: (0, 0)),
            pl.BlockSpec((DIM, 2 * INTER), lambda e, m: (0, 0)),
            pl.BlockSpec((2 * INTER, DIM), lambda e, m: (0, 0)),
        ],
        out_specs=pl.BlockSpec((1, T, DIM), lambda e, m: (0, 0, 0)),
        out_shape=jax.ShapeDtypeStruct((1, T, DIM), jnp.float32),
        scratch_shapes=[
            pltpu.VMEM((T, E), jnp.float32),
            pltpu.VMEM((T, DIM), jnp.bfloat16),
        ],
        compiler_params=pltpu.CompilerParams(
            dimension_semantics=("arbitrary", "arbitrary"),
        ),
        interpret=interpret,
    )(x, gate_w, w1, w2, w3, sw1, sw2, sw3)
    return out


def kernel(x, gate_w, w1, w2, w3, sw1, sw2, sw3):
    out = _moe(x, gate_w, w1, w2, w3, sw1, sw2, sw3)
    aux = jnp.asarray(0.0, dtype=jnp.float32)
    return out, aux


# speedup vs baseline: 2.3114x; 1.0530x over previous
import functools

import jax
import jax.numpy as jnp
from jax import lax
from jax.experimental import pallas as pl
from jax.experimental.pallas import tpu as pltpu
from jax.experimental.pallas import tpu_sc as plsc

DIM = 1024
INTER = 512
E = 16
T = 2048
BM = 256
MT = T // BM
TILE = 128
NTILES = 48
NS = NTILES * TILE
NA = 2 * T
NW = 32


def _silu(v):
    return v * jax.nn.sigmoid(v)


def _bdot(a, b):
    return lax.dot_general(a, b, (((1,), (1,)), ((), ())),
                           preferred_element_type=jnp.float32)


def _gate_body(xr, gwr, sw1r, sw2r, sw3r,
               zr, destr, gwfr, ter,
               exc_s, i_s, carry_s):
    m = pl.program_id(0)
    rows = pl.ds(m * BM, BM)

    @pl.when(m == 0)
    def _():
        carry_s[...] = jnp.zeros((1, E), jnp.float32)

    xf = xr[0]
    logits = _bdot(xf, gwr[...])
    mx = jnp.max(logits, axis=1, keepdims=True)
    p = jnp.exp(logits - mx)
    p = p / jnp.sum(p, axis=1, keepdims=True)
    iota = lax.broadcasted_iota(jnp.int32, (BM, E), 1)
    m1 = jnp.max(p, axis=1, keepdims=True)
    i1 = jnp.min(jnp.where(p == m1, iota, E), axis=1, keepdims=True)
    p2 = jnp.where(iota == i1, -1.0, p)
    m2 = jnp.max(p2, axis=1, keepdims=True)
    i2 = jnp.min(jnp.where(p2 == m2, iota, E), axis=1, keepdims=True)
    maskb = ((iota == i1) | (iota == i2)).astype(jnp.float32)
    ri = lax.broadcasted_iota(jnp.int32, (BM, BM), 0)
    ci = lax.broadcasted_iota(jnp.int32, (BM, BM), 1)
    lower = (ri > ci).astype(jnp.bfloat16)
    excb = lax.dot_general(lower, maskb.astype(jnp.bfloat16),
                           (((1,), (0,)), ((), ())),
                           preferred_element_type=jnp.float32) + carry_s[...]
    exc_s[rows, :] = excb
    carry_s[...] += jnp.sum(maskb, axis=0, keepdims=True)
    i_s[rows, 0:1] = i1
    i_s[rows, 1:2] = i2
    gwfr[0, rows, :] = jnp.broadcast_to(m1, (BM, TILE))
    gwfr[1, rows, :] = jnp.broadcast_to(m2, (BM, TILE))

    xb = xf.astype(jnp.bfloat16)
    h1 = _bdot(xb, sw1r[...].astype(jnp.bfloat16))
    h3 = _bdot(xb, sw3r[...].astype(jnp.bfloat16))
    hh = (_silu(h1) * h3).astype(jnp.bfloat16)
    zr[...] = _bdot(hh, sw2r[...].astype(jnp.bfloat16))

    @pl.when(m == MT - 1)
    def _finalize():
        counts = carry_s[...]
        cpad = (((counts.astype(jnp.int32) + TILE - 1) // TILE)
                * TILE).astype(jnp.float32)
        e1 = lax.broadcasted_iota(jnp.int32, (E, E), 0)
        e2 = lax.broadcasted_iota(jnp.int32, (E, E), 1)
        upper = (e1 < e2).astype(jnp.float32)
        base = lax.dot_general(
            cpad, upper, (((1,), (0,)), ((), ())),
            preferred_element_type=jnp.float32)
        excf = exc_s[...]
        iota_t = lax.broadcasted_iota(jnp.int32, (T, E), 1)
        for k in range(2):
            ik = i_s[:, k:k + 1]
            dk = jnp.sum(jnp.where(iota_t == ik, excf + base, 0.0),
                         axis=1, keepdims=True)
            destr[:, k:k + 1] = dk.astype(jnp.int32)
        ends = base + cpad
        starts = (lax.broadcasted_iota(jnp.int32, (1, 64), 1)
                  * TILE).astype(jnp.float32)
        acc = jnp.zeros((1, 64), jnp.int32)
        for e in range(E):
            acc += (starts >= ends[0:1, e:e + 1]).astype(jnp.int32)
        ter[...] = acc


@functools.partial(jax.jit, static_argnames=("interpret",))
def _gate(x, gate_w, sw1, sw2, sw3, interpret=False):
    return pl.pallas_call(
        _gate_body,
        grid=(MT,),
        in_specs=[
            pl.BlockSpec((1, BM, DIM), lambda m: (0, m, 0)),
            pl.BlockSpec((E, DIM), lambda m: (0, 0)),
            pl.BlockSpec((2 * INTER, DIM), lambda m: (0, 0)),
            pl.BlockSpec((DIM, 2 * INTER), lambda m: (0, 0)),
            pl.BlockSpec((2 * INTER, DIM), lambda m: (0, 0)),
        ],
        out_specs=[
            pl.BlockSpec((BM, DIM), lambda m: (m, 0)),
            pl.BlockSpec((T, 2), lambda m: (0, 0)),
            pl.BlockSpec((2, T, TILE), lambda m: (0, 0, 0)),
            pl.BlockSpec((1, 64), lambda m: (0, 0)),
        ],
        out_shape=[
            jax.ShapeDtypeStruct((T, DIM), jnp.float32),
            jax.ShapeDtypeStruct((T, 2), jnp.int32),
            jax.ShapeDtypeStruct((2, T, TILE), jnp.float32),
            jax.ShapeDtypeStruct((1, 64), jnp.int32),
        ],
        scratch_shapes=[
            pltpu.VMEM((T, E), jnp.float32),
            pltpu.VMEM((T, 2), jnp.int32),
            pltpu.VMEM((1, E), jnp.float32),
        ],
        compiler_params=pltpu.CompilerParams(
            dimension_semantics=("arbitrary",),
        ),
        interpret=interpret,
    )(x, gate_w, sw1, sw2, sw3)


@functools.cache
def _dispatch_sc():
    mesh = plsc.VectorSubcoreMesh(core_axis_name="c", subcore_axis_name="s")

    @functools.partial(
        pl.kernel,
        out_type=[jax.ShapeDtypeStruct((NS, DIM), jnp.float32),
                  jax.ShapeDtypeStruct((NS, TILE), jnp.float32)],
        mesh=mesh,
        scratch_types=[pltpu.VMEM((2, 64), jnp.int32),
                       pltpu.VMEM((2, 64, TILE), jnp.float32),
                       pltpu.VMEM((64, DIM), jnp.float32),
                       pltpu.SemaphoreType.DMA],
    )
    def _body(x_hbm, dest_hbm, gw_hbm, xs_hbm, gws_hbm,
              dest_v, gw_v, rows_v, sem):
        wid = lax.axis_index("s") * 2 + lax.axis_index("c")
        pltpu.sync_copy(dest_hbm.at[wid], dest_v)
        pltpu.sync_copy(gw_hbm.at[wid], gw_v)
        for h in range(2):
            row0 = lax.rem(wid * 128 + h * 64, T)
            pltpu.sync_copy(x_hbm.at[pl.ds(row0, 64)], rows_v)
            pltpu.async_copy(rows_v, xs_hbm.at[dest_v.at[h]], sem).wait()
            pltpu.async_copy(gw_v.at[h], gws_hbm.at[dest_v.at[h]], sem).wait()

    return _body


def _gemm_body(te_ref, xsr, w1r, w2r, w3r, gwsr, ysr):
    i = pl.program_id(0)

    @pl.when(te_ref[i] < E)
    def _():
        xb = xsr[0].astype(jnp.bfloat16)
        h1 = _bdot(xb, w1r[0].astype(jnp.bfloat16))
        h3 = _bdot(xb, w3r[0].astype(jnp.bfloat16))
        g = gwsr[0][:, 0:1]
        hh = (_silu(h1) * h3).astype(jnp.bfloat16)
        ysr[0] = _bdot(hh, w2r[0].astype(jnp.bfloat16)) * g


@functools.partial(jax.jit, static_argnames=("interpret",))
def _gemm(te, xs3, w1, w2, w3, gws3, interpret=False):
    grid_spec = pltpu.PrefetchScalarGridSpec(
        num_scalar_prefetch=1,
        grid=(NTILES,),
        in_specs=[
            pl.BlockSpec((1, TILE, DIM), lambda i, te_ref: (i, 0, 0)),
            pl.BlockSpec((1, INTER, DIM),
                         lambda i, te_ref: (jnp.minimum(te_ref[i], E - 1), 0, 0)),
            pl.BlockSpec((1, DIM, INTER),
                         lambda i, te_ref: (jnp.minimum(te_ref[i], E - 1), 0, 0)),
            pl.BlockSpec((1, INTER, DIM),
                         lambda i, te_ref: (jnp.minimum(te_ref[i], E - 1), 0, 0)),
            pl.BlockSpec((1, TILE, TILE), lambda i, te_ref: (i, 0, 0)),
        ],
        out_specs=pl.BlockSpec((1, TILE, DIM), lambda i, te_ref: (i, 0, 0)),
        scratch_shapes=[],
    )
    return pl.pallas_call(
        _gemm_body,
        grid_spec=grid_spec,
        out_shape=jax.ShapeDtypeStruct((NTILES, TILE, DIM), jnp.float32),
        compiler_params=pltpu.CompilerParams(
            dimension_semantics=("arbitrary",),
        ),
        interpret=interpret,
    )(te, xs3, w1, w2, w3, gws3)


@functools.cache
def _combine_sc():
    mesh = plsc.VectorSubcoreMesh(core_axis_name="c", subcore_axis_name="s")

    @functools.partial(
        pl.kernel,
        out_type=jax.ShapeDtypeStruct((T, DIM), jnp.float32),
        mesh=mesh,
        scratch_types=[pltpu.VMEM((4, 32), jnp.int32),
                       pltpu.VMEM((32, DIM), jnp.float32),
                       pltpu.VMEM((32, DIM), jnp.float32),
                       pltpu.VMEM((32, DIM), jnp.float32),
                       pltpu.SemaphoreType.DMA],
    )
    def _body(z_hbm, ys_hbm, d_hbm, out_hbm, d_v, zc, r1, r2, sem):
        wid = lax.axis_index("s") * 2 + lax.axis_index("c")
        pltpu.sync_copy(d_hbm.at[wid], d_v)
        for h in range(2):
            rows = pl.ds(wid * 64 + h * 32, 32)
            pltpu.sync_copy(z_hbm.at[rows], zc)
            cp1 = pltpu.async_copy(ys_hbm.at[d_v.at[h]], r1, sem)
            cp2 = pltpu.async_copy(ys_hbm.at[d_v.at[2 + h]], r2, sem)
            cp1.wait()
            cp2.wait()
            for r in range(32):
                def body(c, acc):
                    sl = pl.ds(c * 16, 16)
                    zc[r, sl] = zc[r, sl] + r1[r, sl] + r2[r, sl]
                    return acc
                lax.fori_loop(0, DIM // 16, body, 0)
            pltpu.sync_copy(zc, out_hbm.at[rows])

    return _body


def kernel(x, gate_w, w1, w2, w3, sw1, sw2, sw3):
    x2d = x.reshape(T, DIM)
    z, dest_tk, gw_f, te_r = _gate(x, gate_w, sw1, sw2, sw3)
    dest_a = dest_tk.T.reshape(NW, 2, 64)
    gw_a = gw_f.reshape(NW, 2, 64, TILE)
    xs, gws = _dispatch_sc()(x2d, dest_a, gw_a)
    te = te_r.reshape(64)
    ys = _gemm(te, xs.reshape(NTILES, TILE, DIM), w1, w2, w3,
               gws.reshape(NTILES, TILE, TILE))
    d_all = jnp.concatenate([dest_tk[:, 0].reshape(NW, 2, 32),
                             dest_tk[:, 1].reshape(NW, 2, 32)], axis=1)
    out2d = _combine_sc()(z, ys.reshape(NS, DIM), d_all)
    out = out2d.reshape(1, T, DIM)
    aux = jnp.asarray(0.0, dtype=jnp.float32)
    return out, aux
